# baseline (speedup 1.0000x reference)
import functools

import jax
import jax.numpy as jnp
from jax import lax
from jax.experimental import pallas as pl
from jax.experimental.pallas import tpu as pltpu
from jax.experimental.pallas import tpu_sc as plsc

BATCH = 16384
ATOMS = 200
NUM_SPECIES = 8

NC = 2
NS = 16
LANES = 16
NW = NC * NS
ROWS = BATCH // NW


def _sc_body(species_hbm, energies_hbm, table_hbm, out_hbm,
             spec_v, en_v, tab_v, out_v):
    wid = lax.axis_index("s") * NC + lax.axis_index("c")
    base = wid * ROWS

    pltpu.sync_copy(table_hbm, tab_v)
    pltpu.sync_copy(energies_hbm.at[pl.ds(base, ROWS)], en_v)
    pltpu.sync_copy(species_hbm.at[pl.ds(base * ATOMS, ROWS * ATOMS)], spec_v)

    lanes = lax.iota(jnp.int32, LANES)
    for g in range(ROWS // LANES):
        row0 = g * LANES
        row_flat = (row0 + lanes) * ATOMS

        def atom_body(a, acc):
            sv = plsc.load_gather(spec_v, [row_flat + a])
            ev = plsc.load_gather(tab_v, [sv])
            return acc + ev

        acc = lax.fori_loop(0, ATOMS, atom_body,
                            jnp.zeros((LANES,), jnp.float32))
        out_v[pl.ds(row0, LANES)] = acc + en_v[pl.ds(row0, LANES)]

    pltpu.sync_copy(out_v, out_hbm.at[pl.ds(base, ROWS)])


@functools.partial(jax.jit)
def _sc_shift(species, energies, self_energies):
    mesh = plsc.VectorSubcoreMesh(core_axis_name="c", subcore_axis_name="s")
    f = pl.kernel(
        _sc_body,
        out_type=jax.ShapeDtypeStruct((BATCH,), jnp.float32),
        mesh=mesh,
        compiler_params=pltpu.CompilerParams(needs_layout_passes=False),
        scratch_types=[
            pltpu.VMEM((ROWS * ATOMS,), jnp.int32),
            pltpu.VMEM((ROWS,), jnp.float32),
            pltpu.VMEM((NUM_SPECIES,), jnp.float32),
            pltpu.VMEM((ROWS,), jnp.float32),
        ],
    )
    return f(species.reshape(-1), energies, self_energies)


def kernel(species, energies, self_energies):
    shifted = _sc_shift(species, energies, self_energies)
    return (species, shifted)


# speedup vs baseline: 1.4940x; 1.4940x over previous
import functools

import numpy as np
import jax
import jax.numpy as jnp
from jax import lax
from jax.experimental import pallas as pl
from jax.experimental.pallas import tpu as pltpu
from jax.experimental.pallas import tpu_sc as plsc

BATCH = 16384
ATOMS = 200
NUM_SPECIES = 8

NC = 2
NS = 16
LANES = 16
NW = NC * NS
ROWS = BATCH // NW
GROUPS = ROWS // LANES
GROUP_WORDS = LANES * ATOMS

def _lookup(tab_reg, sv):
    return tab_reg.at[sv].get(mode="promise_in_bounds")


def _sc_body(species_hbm, energies_hbm, table_hbm, out_hbm,
             spec_v, en_v, tab_v, acc_v, out_v):
    wid = lax.axis_index("s") * NC + lax.axis_index("c")
    base = wid * ROWS

    lanes = lax.iota(jnp.int32, LANES)
    zero = jnp.where(lanes < 0, 1.0, 0.0).astype(jnp.float32)

    tab_v[...] = zero
    pltpu.sync_copy(table_hbm, tab_v.at[pl.ds(0, NUM_SPECIES)])
    pltpu.sync_copy(energies_hbm.at[pl.ds(base, ROWS)], en_v)
    pltpu.sync_copy(species_hbm.at[pl.ds(base * ATOMS, ROWS * ATOMS)], spec_v)

    tab_reg = tab_v[...]
    lo_mask = lanes < 8
    rot_store = [r * LANES + ((lanes + r) & (LANES - 1))
                 for r in range(LANES)]
    diag_read = [lanes * LANES + ((lanes + j) & (LANES - 1))
                 for j in range(LANES)]

    def group_body(g, carry):
        fb = pl.multiple_of(g * GROUP_WORDS, GROUP_WORDS)
        for p in range(8):
            sb = p * 400
            acc_a = zero
            for j in range(12):
                sv = spec_v[pl.ds(fb + sb + 16 * j, LANES)]
                acc_a = acc_a + _lookup(tab_reg, sv)
            sv_m = spec_v[pl.ds(fb + sb + 192, LANES)]
            mid = _lookup(tab_reg, sv_m)
            acc_b = zero
            for j in range(13, 25):
                sv = spec_v[pl.ds(fb + sb + 16 * j, LANES)]
                acc_b = acc_b + _lookup(tab_reg, sv)
            acc_a = acc_a + jnp.where(lo_mask, mid, zero)
            acc_b = acc_b + jnp.where(lo_mask, zero, mid)
            plsc.store_scatter(acc_v, [rot_store[2 * p]], acc_a)
            plsc.store_scatter(acc_v, [rot_store[2 * p + 1]], acc_b)
        tot = zero
        for j in range(LANES):
            tot = tot + plsc.load_gather(acc_v, [diag_read[j]])
        row0 = pl.multiple_of(g * LANES, LANES)
        out_v[pl.ds(row0, LANES)] = tot + en_v[pl.ds(row0, LANES)]
        return carry

    lax.fori_loop(0, GROUPS, group_body, 0)
    pltpu.sync_copy(out_v, out_hbm.at[pl.ds(base, ROWS)])


@functools.partial(jax.jit)
def _sc_shift(species, energies, self_energies):
    mesh = plsc.VectorSubcoreMesh(core_axis_name="c", subcore_axis_name="s")
    f = pl.kernel(
        _sc_body,
        out_type=jax.ShapeDtypeStruct((BATCH,), jnp.float32),
        mesh=mesh,
        compiler_params=pltpu.CompilerParams(needs_layout_passes=False),
        scratch_types=[
            pltpu.VMEM((ROWS * ATOMS,), jnp.int32),
            pltpu.VMEM((ROWS,), jnp.float32),
            pltpu.VMEM((LANES,), jnp.float32),
            pltpu.VMEM((LANES * LANES,), jnp.float32),
            pltpu.VMEM((ROWS,), jnp.float32),
        ],
    )
    return f(species.reshape(-1), energies, self_energies)


def kernel(species, energies, self_energies):
    shifted = _sc_shift(species, energies, self_energies)
    return (species, shifted)


# speedup vs baseline: 2.2098x; 1.4790x over previous
import functools

import jax
import jax.numpy as jnp
from jax import lax
from jax.experimental import pallas as pl
from jax.experimental.pallas import tpu as pltpu

BATCH = 16384
ATOMS = 200
NUM_SPECIES = 8

BR = 512


def _tc_body(tab_ref, spec_ref, en_ref, out_ref):
    x = spec_ref[...]
    acc = jnp.zeros(x.shape, jnp.float32)
    for k in range(NUM_SPECIES):
        acc = jnp.where(x == k, tab_ref[k], acc)
    out_ref[...] = en_ref[...] + jnp.sum(acc, axis=1)


@functools.partial(jax.jit)
def _tc_shift(species, energies, self_energies):
    grid = (BATCH // BR,)
    return pl.pallas_call(
        _tc_body,
        grid=grid,
        in_specs=[
            pl.BlockSpec(memory_space=pltpu.SMEM),
            pl.BlockSpec((BR, ATOMS), lambda i: (i, 0)),
            pl.BlockSpec((BR,), lambda i: (i,)),
        ],
        out_specs=pl.BlockSpec((BR,), lambda i: (i,)),
        out_shape=jax.ShapeDtypeStruct((BATCH,), jnp.float32),
        compiler_params=pltpu.CompilerParams(
            dimension_semantics=("arbitrary",)),
    )(self_energies, species, energies)


def kernel(species, energies, self_energies):
    shifted = _tc_shift(species, energies, self_energies)
    return (species, shifted)
